# baseline (speedup 1.0000x reference)
import jax
import jax.numpy as jnp
import numpy as np
from jax.experimental import pallas as pl
from jax.experimental.pallas import tpu as pltpu

B, T, D, K_REP, CS, H = 4, 2048, 1024, 16, 48, 8
CAP = 256
DH = D // H
TB = 256

_MIN32 = np.int32(-2147483648)
_M31 = np.int32(0x7FFFFFFF)


def _dotT(x, w):
    return jax.lax.dot_general(x, w, (((1,), (1,)), ((), ())),
                               preferred_element_type=jnp.float32)


def _mlp(q, W1, b1, g, be, W2, b2):
    h = _dotT(q, W1) + b1
    m = h.mean(-1, keepdims=True)
    v = ((h - m) ** 2).mean(-1, keepdims=True)
    h = (h - m) / jnp.sqrt(v + 1e-5) * g + be
    h = jnp.maximum(h, 0.0)
    return _dotT(h, W2) + b2


def _prepdist_body(q_ref, mw1, mb1, mg, mbe, mw2, mb2,
                   sw1, sb1, sg_, sbe, sw2, sb2, v_ref,
                   mu_ref, sig_ref, d_ref, idx_ref, rows_ref,
                   mu_s, rs_s, ds_s):
    i = pl.program_id(0)

    @pl.when(i == 0)
    def _():
        q = q_ref[...]
        mu = _mlp(q, mw1[...], mb1[...], mg[...], mbe[...], mw2[...], mb2[...])
        s = _mlp(q, sw1[...], sb1[...], sg_[...], sbe[...], sw2[...], sb2[...])
        sig = jnp.maximum(s, 0.0) + jnp.log1p(jnp.exp(-jnp.abs(s))) + 1e-6
        mu_ref[...] = mu
        sig_ref[...] = sig
        mu_s[...] = mu
        rs_s[...] = 1.0 / sig

    c = v_ref[...] - mu_s[...][:, None, :]
    dblk = jnp.sum(c * c * rs_s[...][:, None, :], axis=-1)
    d_ref[...] = dblk
    ds_s[:, pl.ds(i * TB, TB)] = dblk

    @pl.when(i == T // TB - 1)
    def _():
        idx = _select_compute(ds_s[...])
        idx_ref[...] = idx
        rows_ref[...] = idx + jax.lax.broadcasted_iota(
            jnp.int32, (B, K_REP), 0) * T


def _prefix_sum(x, n):
    s = 1
    while s < n:
        x = x + jnp.concatenate(
            [jnp.zeros((B, s), jnp.int32), x[:, :n - s]], axis=1)
        s *= 2
    return x


def _select_compute(d):
    u = jax.lax.bitcast_convert_type(d, jnp.int32)

    umin = jnp.min(u, axis=1, keepdims=True)
    umax = jnp.max(u, axis=1, keepdims=True)
    rngf = (umax - umin).astype(jnp.float32)
    e = jax.lax.shift_right_logical(
        jax.lax.bitcast_convert_type(rngf, jnp.int32), 23) & 255
    sh = jnp.clip(158 - e, 0, 31)
    u2 = u - umin
    dig = jax.lax.shift_right_logical(
        jnp.left_shift(u2, sh), 24) & 255

    digT = jnp.transpose(dig)
    dcols = [jax.lax.broadcast_in_dim(digT[:, b:b + 1], (T, 256), (0, 1))
             for b in range(B)]
    DT = jnp.concatenate(dcols, axis=1)
    iota_v = jax.lax.broadcasted_iota(jnp.int32, (T, B * 256), 1) & 255
    OH = (DT == iota_v).astype(jnp.float32)
    ones = jnp.ones((1, T), jnp.float32)
    cntf = jax.lax.dot_general(ones, OH, (((1,), (0,)), ((), ())),
                               precision=jax.lax.Precision.HIGHEST,
                               preferred_element_type=jnp.float32)
    cnt = jnp.concatenate(
        [cntf[0:1, b * 256:(b + 1) * 256] for b in range(B)],
        axis=0).astype(jnp.int32)
    cum = _prefix_sum(cnt, 256)

    kmed = (T - 1) // 2
    lo_bin = jnp.sum(jnp.where(cum <= kmed - (CS - 1), 1, 0),
                     axis=1, keepdims=True)
    hi_bin = jnp.sum(jnp.where(cum <= kmed + (CS - 1), 1, 0),
                     axis=1, keepdims=True)
    iota256 = jax.lax.broadcasted_iota(jnp.int32, (B, 256), 1)
    n_below = jnp.sum(jnp.where(iota256 < lo_bin, cnt, 0),
                      axis=1, keepdims=True)

    m_all = (dig >= lo_bin) & (dig <= hi_bin)
    mi = jnp.where(m_all, 1, 0)
    n_all = jnp.sum(mi, axis=1, keepdims=True)
    p = _prefix_sum(mi, T) - 1
    sel = jnp.where(m_all, p, -1)
    selT = jnp.transpose(sel)
    cols = [jax.lax.broadcast_in_dim(selT[:, b:b + 1], (T, CAP), (0, 1))
            for b in range(B)]
    PT = jnp.concatenate(cols, axis=1)
    iota_r = jax.lax.broadcasted_iota(jnp.int32, (T, B * CAP), 1) & (CAP - 1)
    WT = (PT == iota_r).astype(jnp.float32)
    iota_tf = jax.lax.broadcasted_iota(jnp.int32, (1, T), 1).astype(jnp.float32)
    lhs = jnp.concatenate([d, iota_tf], axis=0)
    packed = jax.lax.dot_general(lhs, WT, (((1,), (0,)), ((), ())),
                                 precision=jax.lax.Precision.HIGHEST,
                                 preferred_element_type=jnp.float32)
    cd = jnp.concatenate(
        [packed[b:b + 1, b * CAP:(b + 1) * CAP] for b in range(B)], axis=0)
    cidx_f = jnp.concatenate(
        [packed[B:B + 1, b * CAP:(b + 1) * CAP] for b in range(B)], axis=0)
    cidx = cidx_f.astype(jnp.int32)

    iota_cap = jax.lax.broadcasted_iota(jnp.int32, (B, CAP), 1)
    valid = iota_cap < n_all
    jio = jax.lax.broadcasted_iota(jnp.int32, (CAP, CAP), 0)
    iio = jax.lax.broadcasted_iota(jnp.int32, (CAP, CAP), 1)

    def pair_rank(cu):
        cuT = jnp.transpose(cu)
        rows = []
        for b in range(B):
            aj = jax.lax.broadcast_in_dim(cuT[:, b:b + 1], (CAP, CAP), (0, 1))
            ai = jax.lax.broadcast_in_dim(cu[b:b + 1, :], (CAP, CAP), (0, 1))
            before = (aj < ai) | ((aj == ai) & (jio < iio))
            rows.append(jnp.sum(jnp.where(before, 1, 0), axis=0,
                                keepdims=True))
        return jnp.concatenate(rows, axis=0)

    cdu = jax.lax.bitcast_convert_type(cd, jnp.int32)
    rank1 = pair_rank(jnp.where(valid, cdu, _M31))
    med_rank = kmed - n_below
    target = jnp.sum(jnp.where(rank1 == med_rank, cd, 0.0),
                     axis=1, keepdims=True)

    cd2t = jnp.abs(cd - target)
    cu2 = jax.lax.bitcast_convert_type(cd2t, jnp.int32)
    rank = pair_rank(jnp.where(valid, cu2, _M31))

    key0 = jnp.where(rank == 0, 2048 + cidx, cidx)
    first = jnp.max(key0, axis=1, keepdims=True) - 2048
    invalid = rank >= CS
    removed = (rank == 0) | invalid
    md = jnp.abs(cidx - first)
    rk_enc = jnp.left_shift(CAP - 1 - rank, 11)
    sel_list = [first]
    for _ in range(K_REP - 1):
        score = jnp.where(removed, 0, md)
        key = jnp.left_shift(score, 19) | rk_enc | cidx
        mx = jnp.max(key, axis=1, keepdims=True)
        new = mx & 2047
        win_rk = (CAP - 1) - (jnp.right_shift(mx, 11) & (CAP - 1))
        removed = removed | (rank == win_rk)
        md = jnp.minimum(md, jnp.abs(cidx - new))
        sel_list.append(new)
    return jnp.concatenate(sel_list, axis=1)


def _mha_body(rows_ref, video_ref, win, bin_, wout, bout,
              out_ref, aw_ref, rep_ref, sem):
    copies = [
        pltpu.make_async_copy(video_ref.at[pl.ds(rows_ref[j], 1), :],
                              rep_ref.at[pl.ds(j, 1), :], sem)
        for j in range(B * K_REP)
    ]
    for c in copies:
        c.start()
    for c in copies:
        c.wait()
    x = rep_ref[...]
    qkv = _dotT(x, win[...]) + bin_[...]
    scale = np.float32(1.0 / np.sqrt(DH))
    ir = jax.lax.broadcasted_iota(jnp.int32, (H * K_REP, D), 0)
    ie = jax.lax.broadcasted_iota(jnp.int32, (H * K_REP, D), 1)
    hmask = jnp.right_shift(ie, 7) == jnp.right_shift(ir, 4)
    outs, aws = [], []
    for b in range(B):
        qkvb = qkv[b * K_REP:(b + 1) * K_REP]
        qb = qkvb[:, 0:D]
        kb = qkvb[:, D:2 * D]
        vb = qkvb[:, 2 * D:3 * D]
        krep = jnp.concatenate([kb] * H, axis=0)
        vrep = jnp.concatenate([vb] * H, axis=0)
        kstack = jnp.where(hmask, krep, 0.0)
        vstack = jnp.where(hmask, vrep, 0.0)
        st = jax.lax.dot_general(kstack, qb, (((1,), (1,)), ((), ())),
                                 preferred_element_type=jnp.float32) * scale
        segs, acc_t = [], jnp.zeros((K_REP, K_REP), jnp.float32)
        for h in range(H):
            seg = st[h * K_REP:(h + 1) * K_REP, :]
            m = jnp.max(seg, axis=0, keepdims=True)
            e = jnp.exp(seg - m)
            a = e / jnp.sum(e, axis=0, keepdims=True)
            segs.append(a)
            acc_t = acc_t + a
        at = jnp.concatenate(segs, axis=0)
        o = jax.lax.dot_general(at, vstack, (((0,), (0,)), ((), ())),
                                preferred_element_type=jnp.float32)
        outs.append(_dotT(o, wout[...]) + bout[...])
        aws.append(jnp.transpose(acc_t) * np.float32(1.0 / H))
    out_ref[...] = jnp.stack(outs, axis=0)
    aw_ref[...] = jnp.stack(aws, axis=0)


def kernel(video_features, query_embedding, mu_W1, mu_b1, mu_g, mu_be, mu_W2,
           mu_b2, sg_W1, sg_b1, sg_g, sg_be, sg_W2, sg_b2, attn_in_W,
           attn_in_b, attn_out_W, attn_out_b):
    f32 = jnp.float32
    wspec = pl.BlockSpec(memory_space=pltpu.MemorySpace.VMEM)
    mu_q, sigma, dist, idx, rows = pl.pallas_call(
        _prepdist_body,
        grid=(T // TB,),
        in_specs=[wspec] * 13 + [pl.BlockSpec((B, TB, D), lambda i: (0, i, 0))],
        out_specs=(pl.BlockSpec((B, D), lambda i: (0, 0)),
                   pl.BlockSpec((B, D), lambda i: (0, 0)),
                   pl.BlockSpec((B, TB), lambda i: (0, i)),
                   pl.BlockSpec((B, K_REP), lambda i: (0, 0)),
                   pl.BlockSpec((B, K_REP), lambda i: (0, 0))),
        out_shape=(jax.ShapeDtypeStruct((B, D), f32),
                   jax.ShapeDtypeStruct((B, D), f32),
                   jax.ShapeDtypeStruct((B, T), f32),
                   jax.ShapeDtypeStruct((B, K_REP), jnp.int32),
                   jax.ShapeDtypeStruct((B, K_REP), jnp.int32)),
        scratch_shapes=[pltpu.VMEM((B, D), f32), pltpu.VMEM((B, D), f32),
                        pltpu.VMEM((B, T), f32)],
    )(query_embedding, mu_W1, mu_b1, mu_g, mu_be, mu_W2, mu_b2,
      sg_W1, sg_b1, sg_g, sg_be, sg_W2, sg_b2, video_features)

    video_r = video_features.reshape(B * T, D)
    refined, attn_w = pl.pallas_call(
        _mha_body,
        in_specs=[pl.BlockSpec(memory_space=pltpu.MemorySpace.SMEM),
                  pl.BlockSpec(memory_space=pltpu.MemorySpace.HBM),
                  wspec, wspec, wspec, wspec],
        out_shape=(jax.ShapeDtypeStruct((B, K_REP, D), f32),
                   jax.ShapeDtypeStruct((B, K_REP, K_REP), f32)),
        scratch_shapes=[pltpu.VMEM((B * K_REP, D), f32),
                        pltpu.SemaphoreType.DMA],
    )(rows.reshape(B * K_REP), video_r,
      attn_in_W, attn_in_b, attn_out_W, attn_out_b)

    return refined, idx, dist, mu_q, sigma, attn_w


# speedup vs baseline: 1.0526x; 1.0526x over previous
import jax
import jax.numpy as jnp
import numpy as np
from jax.experimental import pallas as pl
from jax.experimental.pallas import tpu as pltpu

B, T, D, K_REP, CS, H = 4, 2048, 1024, 16, 48, 8
CAP = 256
DH = D // H
TB = 256

_M31 = np.int32(0x7FFFFFFF)


def _dotT(x, w):
    return jax.lax.dot_general(x, w, (((1,), (1,)), ((), ())),
                               preferred_element_type=jnp.float32)


def _mlp(q, W1, b1, g, be, W2, b2):
    h = _dotT(q, W1) + b1
    m = h.mean(-1, keepdims=True)
    v = ((h - m) ** 2).mean(-1, keepdims=True)
    h = (h - m) / jnp.sqrt(v + 1e-5) * g + be
    h = jnp.maximum(h, 0.0)
    return _dotT(h, W2) + b2


def _prepdist_body(q_ref, mw1, mb1, mg, mbe, mw2, mb2,
                   sw1, sb1, sg_, sbe, sw2, sb2, v_ref,
                   mu_ref, sig_ref, d_ref, idx_ref, rows_ref,
                   mu_s, rs_s, ds_s):
    i = pl.program_id(0)

    @pl.when(i == 0)
    def _():
        q = q_ref[...]
        mu = _mlp(q, mw1[...], mb1[...], mg[...], mbe[...], mw2[...], mb2[...])
        s = _mlp(q, sw1[...], sb1[...], sg_[...], sbe[...], sw2[...], sb2[...])
        sig = jnp.maximum(s, 0.0) + jnp.log1p(jnp.exp(-jnp.abs(s))) + 1e-6
        mu_ref[...] = mu
        sig_ref[...] = sig
        mu_s[...] = mu
        rs_s[...] = 1.0 / sig

    c = v_ref[...] - mu_s[...][:, None, :]
    dblk = jnp.sum(c * c * rs_s[...][:, None, :], axis=-1)
    d_ref[...] = dblk
    ds_s[:, pl.ds(i * TB, TB)] = dblk

    @pl.when(i == T // TB - 1)
    def _():
        idx = _select_compute(ds_s[...])
        idx_ref[...] = idx
        rows_ref[...] = idx + jax.lax.broadcasted_iota(
            jnp.int32, (B, K_REP), 0) * T


def _prefix_sum(x, n):
    s = 1
    while s < n:
        x = x + jnp.concatenate(
            [jnp.zeros((B, s), jnp.int32), x[:, :n - s]], axis=1)
        s *= 2
    return x


def _select_compute(d):
    u = jax.lax.bitcast_convert_type(d, jnp.int32)

    umin = jnp.min(u, axis=1, keepdims=True)
    umax = jnp.max(u, axis=1, keepdims=True)
    rngf = (umax - umin).astype(jnp.float32)
    e = jax.lax.shift_right_logical(
        jax.lax.bitcast_convert_type(rngf, jnp.int32), 23) & 255
    sh = jnp.clip(158 - e, 0, 31)
    u2 = u - umin
    dig = jax.lax.shift_right_logical(
        jnp.left_shift(u2, sh), 24) & 255

    digT = jnp.transpose(dig)
    dcols = [jax.lax.broadcast_in_dim(digT[:, b:b + 1], (T, 256), (0, 1))
             for b in range(B)]
    DT = jnp.concatenate(dcols, axis=1)
    iota_v = jax.lax.broadcasted_iota(jnp.int32, (T, B * 256), 1) & 255
    OH = (DT == iota_v).astype(jnp.bfloat16)
    ones = jnp.ones((1, T), jnp.bfloat16)
    cntf = jax.lax.dot_general(ones, OH, (((1,), (0,)), ((), ())),
                               preferred_element_type=jnp.float32)
    cnt = jnp.concatenate(
        [cntf[0:1, b * 256:(b + 1) * 256] for b in range(B)],
        axis=0).astype(jnp.int32)
    cum = _prefix_sum(cnt, 256)

    kmed = (T - 1) // 2
    lo_bin = jnp.sum(jnp.where(cum <= kmed - (CS - 1), 1, 0),
                     axis=1, keepdims=True)
    hi_bin = jnp.sum(jnp.where(cum <= kmed + (CS - 1), 1, 0),
                     axis=1, keepdims=True)
    iota256 = jax.lax.broadcasted_iota(jnp.int32, (B, 256), 1)
    n_below = jnp.sum(jnp.where(iota256 < lo_bin, cnt, 0),
                      axis=1, keepdims=True)

    m_all = (dig >= lo_bin) & (dig <= hi_bin)
    mi = jnp.where(m_all, 1, 0)
    n_all = jnp.sum(mi, axis=1, keepdims=True)
    p = _prefix_sum(mi, T) - 1
    sel = jnp.where(m_all, p, -1)
    selT = jnp.transpose(sel)
    cols = [jax.lax.broadcast_in_dim(selT[:, b:b + 1], (T, CAP), (0, 1))
            for b in range(B)]
    PT = jnp.concatenate(cols, axis=1)
    iota_r = jax.lax.broadcasted_iota(jnp.int32, (T, B * CAP), 1) & (CAP - 1)
    WT = (PT == iota_r).astype(jnp.float32)
    iota_tf = jax.lax.broadcasted_iota(jnp.int32, (1, T), 1).astype(jnp.float32)
    lhs = jnp.concatenate([d, iota_tf], axis=0)
    packed = jax.lax.dot_general(lhs, WT, (((1,), (0,)), ((), ())),
                                 precision=jax.lax.Precision.HIGHEST,
                                 preferred_element_type=jnp.float32)
    cd = jnp.concatenate(
        [packed[b:b + 1, b * CAP:(b + 1) * CAP] for b in range(B)], axis=0)
    cidx_f = jnp.concatenate(
        [packed[B:B + 1, b * CAP:(b + 1) * CAP] for b in range(B)], axis=0)
    cidx = cidx_f.astype(jnp.int32)

    iota_cap = jax.lax.broadcasted_iota(jnp.int32, (B, CAP), 1)
    valid = iota_cap < n_all
    jio = jax.lax.broadcasted_iota(jnp.int32, (CAP, CAP), 0)
    iio = jax.lax.broadcasted_iota(jnp.int32, (CAP, CAP), 1)

    def pair_rank(cu):
        cuT = jnp.transpose(cu)
        rows = []
        for b in range(B):
            aj = jax.lax.broadcast_in_dim(cuT[:, b:b + 1], (CAP, CAP), (0, 1))
            ai = jax.lax.broadcast_in_dim(cu[b:b + 1, :], (CAP, CAP), (0, 1))
            before = (aj < ai) | ((aj == ai) & (jio < iio))
            rows.append(jnp.sum(jnp.where(before, 1, 0), axis=0,
                                keepdims=True))
        return jnp.concatenate(rows, axis=0)

    cdu = jax.lax.bitcast_convert_type(cd, jnp.int32)
    rank1 = pair_rank(jnp.where(valid, cdu, _M31))
    med_rank = kmed - n_below
    target = jnp.sum(jnp.where(rank1 == med_rank, cd, 0.0),
                     axis=1, keepdims=True)

    cd2t = jnp.abs(cd - target)
    cu2 = jax.lax.bitcast_convert_type(cd2t, jnp.int32)
    rank = pair_rank(jnp.where(valid, cu2, _M31))

    key0 = jnp.where(rank == 0, 2048 + cidx, cidx)
    first = jnp.max(key0, axis=1, keepdims=True) - 2048
    invalid = rank >= CS
    removed = (rank == 0) | invalid
    md = jnp.abs(cidx - first)
    rk_enc = jnp.left_shift(CAP - 1 - rank, 11)
    sel_list = [first]
    for _ in range(K_REP - 1):
        score = jnp.where(removed, 0, md)
        key = jnp.left_shift(score, 19) | rk_enc | cidx
        mx = jnp.max(key, axis=1, keepdims=True)
        new = mx & 2047
        win_rk = (CAP - 1) - (jnp.right_shift(mx, 11) & (CAP - 1))
        removed = removed | (rank == win_rk)
        md = jnp.minimum(md, jnp.abs(cidx - new))
        sel_list.append(new)
    return jnp.concatenate(sel_list, axis=1)


def _mha_body(rows_ref, video_ref, win, bin_, wout, bout,
              out_ref, aw_ref, rep_ref, sem):
    copies = [
        pltpu.make_async_copy(video_ref.at[pl.ds(rows_ref[j], 1), :],
                              rep_ref.at[pl.ds(j, 1), :], sem)
        for j in range(B * K_REP)
    ]
    for c in copies:
        c.start()
    for c in copies:
        c.wait()
    x = rep_ref[...]
    qkv = _dotT(x, win[...]) + bin_[...]
    scale = np.float32(1.0 / np.sqrt(DH))
    ir = jax.lax.broadcasted_iota(jnp.int32, (H * K_REP, D), 0)
    ie = jax.lax.broadcasted_iota(jnp.int32, (H * K_REP, D), 1)
    hmask = jnp.right_shift(ie, 7) == jnp.right_shift(ir, 4)
    outs, aws = [], []
    for b in range(B):
        qkvb = qkv[b * K_REP:(b + 1) * K_REP]
        qb = qkvb[:, 0:D]
        kb = qkvb[:, D:2 * D]
        vb = qkvb[:, 2 * D:3 * D]
        krep = jnp.concatenate([kb] * H, axis=0)
        vrep = jnp.concatenate([vb] * H, axis=0)
        kstack = jnp.where(hmask, krep, 0.0)
        vstack = jnp.where(hmask, vrep, 0.0)
        st = jax.lax.dot_general(kstack, qb, (((1,), (1,)), ((), ())),
                                 preferred_element_type=jnp.float32) * scale
        segs, acc_t = [], jnp.zeros((K_REP, K_REP), jnp.float32)
        for h in range(H):
            seg = st[h * K_REP:(h + 1) * K_REP, :]
            m = jnp.max(seg, axis=0, keepdims=True)
            e = jnp.exp(seg - m)
            a = e / jnp.sum(e, axis=0, keepdims=True)
            segs.append(a)
            acc_t = acc_t + a
        at = jnp.concatenate(segs, axis=0)
        o = jax.lax.dot_general(at, vstack, (((0,), (0,)), ((), ())),
                                preferred_element_type=jnp.float32)
        outs.append(_dotT(o, wout[...]) + bout[...])
        aws.append(jnp.transpose(acc_t) * np.float32(1.0 / H))
    out_ref[...] = jnp.stack(outs, axis=0)
    aw_ref[...] = jnp.stack(aws, axis=0)


def kernel(video_features, query_embedding, mu_W1, mu_b1, mu_g, mu_be, mu_W2,
           mu_b2, sg_W1, sg_b1, sg_g, sg_be, sg_W2, sg_b2, attn_in_W,
           attn_in_b, attn_out_W, attn_out_b):
    f32 = jnp.float32
    wspec = pl.BlockSpec(memory_space=pltpu.MemorySpace.VMEM)
    mu_q, sigma, dist, idx, rows = pl.pallas_call(
        _prepdist_body,
        grid=(T // TB,),
        in_specs=[wspec] * 13 + [pl.BlockSpec((B, TB, D), lambda i: (0, i, 0))],
        out_specs=(pl.BlockSpec((B, D), lambda i: (0, 0)),
                   pl.BlockSpec((B, D), lambda i: (0, 0)),
                   pl.BlockSpec((B, TB), lambda i: (0, i)),
                   pl.BlockSpec((B, K_REP), lambda i: (0, 0)),
                   pl.BlockSpec((B, K_REP), lambda i: (0, 0))),
        out_shape=(jax.ShapeDtypeStruct((B, D), f32),
                   jax.ShapeDtypeStruct((B, D), f32),
                   jax.ShapeDtypeStruct((B, T), f32),
                   jax.ShapeDtypeStruct((B, K_REP), jnp.int32),
                   jax.ShapeDtypeStruct((B, K_REP), jnp.int32)),
        scratch_shapes=[pltpu.VMEM((B, D), f32), pltpu.VMEM((B, D), f32),
                        pltpu.VMEM((B, T), f32)],
    )(query_embedding, mu_W1, mu_b1, mu_g, mu_be, mu_W2, mu_b2,
      sg_W1, sg_b1, sg_g, sg_be, sg_W2, sg_b2, video_features)

    video_r = video_features.reshape(B * T, D)
    refined, attn_w = pl.pallas_call(
        _mha_body,
        in_specs=[pl.BlockSpec(memory_space=pltpu.MemorySpace.SMEM),
                  pl.BlockSpec(memory_space=pltpu.MemorySpace.HBM),
                  wspec, wspec, wspec, wspec],
        out_shape=(jax.ShapeDtypeStruct((B, K_REP, D), f32),
                   jax.ShapeDtypeStruct((B, K_REP, K_REP), f32)),
        scratch_shapes=[pltpu.VMEM((B * K_REP, D), f32),
                        pltpu.SemaphoreType.DMA],
    )(rows.reshape(B * K_REP), video_r,
      attn_in_W, attn_in_b, attn_out_W, attn_out_b)

    return refined, idx, dist, mu_q, sigma, attn_w
